# baseline (speedup 1.0000x reference)
import functools

import jax
import jax.numpy as jnp
from jax import lax
from jax.experimental import pallas as pl
from jax.experimental.pallas import tpu as pltpu
from jax.experimental.pallas import tpu_sc as plsc

B, P, FD, NVOX = 4, 25000, 256, 80000


_BV = 10240
_GRID = (NVOX + _BV - 1) // _BV
_NVPAD = _GRID * _BV


def _matvec_body(x_ref, w_ref, b_ref, o_ref):
    r = jax.lax.dot_general(
        w_ref[:], x_ref[:],
        dimension_numbers=(((1,), (1,)), ((), ())),
        preferred_element_type=jnp.float32,
    )
    o_ref[:] = r[0] + b_ref[0, 0]


def _voxel_scores(feats, W, b2d):
    return pl.pallas_call(
        _matvec_body,
        grid=(_GRID,),
        in_specs=[
            pl.BlockSpec((_BV, FD), lambda i: (i, 0)),
            pl.BlockSpec((1, FD), lambda i: (0, 0)),
            pl.BlockSpec((1, 1), lambda i: (0, 0)),
        ],
        out_specs=pl.BlockSpec((_BV,), lambda i: (i,)),
        out_shape=jax.ShapeDtypeStruct((_NVPAD,), jnp.float32),
    )(feats, W, b2d)


_CH = 3200
_CPR = 8
_LAST_COL = P - _CH


@functools.lru_cache(maxsize=1)
def _make_sc_gather():
    mesh = plsc.VectorSubcoreMesh(core_axis_name="c", subcore_axis_name="s")

    @functools.partial(
        pl.kernel,
        mesh=mesh,
        out_type=jax.ShapeDtypeStruct((B * P,), jnp.float32),
        scratch_types=[
            pltpu.VMEM((_CH,), jnp.int32),
            pltpu.VMEM((_CH,), jnp.float32),
            pltpu.SemaphoreType.DMA,
        ],
    )
    def gather_k(scores_hbm, idx_hbm, out_hbm, idx_v, vals_v, sem):
        wid = lax.axis_index("s") * 2 + lax.axis_index("c")
        base = jnp.minimum(wid * _CH, B * P - _CH)
        pltpu.sync_copy(idx_hbm.at[pl.ds(base, _CH)], idx_v)
        pltpu.async_copy(scores_hbm.at[idx_v], vals_v, sem).wait()
        pltpu.sync_copy(vals_v, out_hbm.at[pl.ds(base, _CH)])

    return gather_k


def kernel(feats_backbone, quantize2original, W, b):
    scores = _voxel_scores(feats_backbone, W, b.reshape(1, 1))
    return _make_sc_gather()(scores, quantize2original).reshape(B, P)


# speedup vs baseline: 1.0081x; 1.0081x over previous
import functools

import jax
import jax.numpy as jnp
from jax import lax
from jax.experimental import pallas as pl
from jax.experimental.pallas import tpu as pltpu
from jax.experimental.pallas import tpu_sc as plsc

B, P, FD, NVOX = 4, 25000, 256, 80000


_BV = 8192
_GRID = (NVOX + _BV - 1) // _BV
_NVPAD = _GRID * _BV


def _matvec_body(x_ref, w_ref, b_ref, o_ref):
    r = jax.lax.dot_general(
        w_ref[:], x_ref[:],
        dimension_numbers=(((1,), (1,)), ((), ())),
        preferred_element_type=jnp.float32,
    )
    o_ref[:] = r[0] + b_ref[0, 0]


def _voxel_scores(feats, W, b2d):
    return pl.pallas_call(
        _matvec_body,
        grid=(_GRID,),
        in_specs=[
            pl.BlockSpec((_BV, FD), lambda i: (i, 0)),
            pl.BlockSpec((1, FD), lambda i: (0, 0)),
            pl.BlockSpec((1, 1), lambda i: (0, 0)),
        ],
        out_specs=pl.BlockSpec((_BV,), lambda i: (i,)),
        out_shape=jax.ShapeDtypeStruct((_NVPAD,), jnp.float32),
    )(feats, W, b2d)


_CH = 3200
_CPR = 8
_LAST_COL = P - _CH


@functools.lru_cache(maxsize=1)
def _make_sc_gather():
    mesh = plsc.VectorSubcoreMesh(core_axis_name="c", subcore_axis_name="s")

    @functools.partial(
        pl.kernel,
        mesh=mesh,
        out_type=jax.ShapeDtypeStruct((B * P,), jnp.float32),
        scratch_types=[
            pltpu.VMEM((_CH,), jnp.int32),
            pltpu.VMEM((_CH,), jnp.float32),
            pltpu.SemaphoreType.DMA,
            pltpu.SemaphoreType.DMA,
            pltpu.SemaphoreType.DMA,
            pltpu.SemaphoreType.DMA,
        ],
    )
    def gather_k(scores_hbm, idx_hbm, out_hbm, idx_v, vals_v,
                 sem1, sem2, sem3, sem4):
        wid = lax.axis_index("s") * 2 + lax.axis_index("c")
        base = jnp.minimum(wid * _CH, B * P - _CH)
        h = _CH // 2
        pltpu.sync_copy(idx_hbm.at[pl.ds(base, h)], idx_v.at[pl.ds(0, h)])
        g1 = pltpu.async_copy(scores_hbm.at[idx_v.at[pl.ds(0, h)]],
                              vals_v.at[pl.ds(0, h)], sem1)
        pltpu.sync_copy(idx_hbm.at[pl.ds(base + h, h)],
                        idx_v.at[pl.ds(h, h)])
        g2 = pltpu.async_copy(scores_hbm.at[idx_v.at[pl.ds(h, h)]],
                              vals_v.at[pl.ds(h, h)], sem2)
        g1.wait()
        o1 = pltpu.async_copy(vals_v.at[pl.ds(0, h)],
                              out_hbm.at[pl.ds(base, h)], sem3)
        g2.wait()
        o2 = pltpu.async_copy(vals_v.at[pl.ds(h, h)],
                              out_hbm.at[pl.ds(base + h, h)], sem4)
        o1.wait()
        o2.wait()

    return gather_k


def kernel(feats_backbone, quantize2original, W, b):
    scores = _voxel_scores(feats_backbone, W, b.reshape(1, 1))
    return _make_sc_gather()(scores, quantize2original).reshape(B, P)


# speedup vs baseline: 1.0154x; 1.0072x over previous
import functools

import jax
import jax.numpy as jnp
from jax import lax
from jax.experimental import pallas as pl
from jax.experimental.pallas import tpu as pltpu
from jax.experimental.pallas import tpu_sc as plsc

B, P, FD, NVOX = 4, 25000, 256, 80000


_BV = 8192
_GRID = (NVOX + _BV - 1) // _BV
_NVPAD = _GRID * _BV


def _matvec_body(x_ref, w_ref, b_ref, o_ref):
    r = jax.lax.dot_general(
        w_ref[:], x_ref[:],
        dimension_numbers=(((1,), (1,)), ((), ())),
        preferred_element_type=jnp.float32,
    )
    o_ref[:] = r[0] + b_ref[0, 0]


def _voxel_scores(feats, W, b2d):
    return pl.pallas_call(
        _matvec_body,
        grid=(_GRID,),
        in_specs=[
            pl.BlockSpec((_BV, FD), lambda i: (i, 0)),
            pl.BlockSpec((1, FD), lambda i: (0, 0)),
            pl.BlockSpec((1, 1), lambda i: (0, 0)),
        ],
        out_specs=pl.BlockSpec((_BV,), lambda i: (i,)),
        out_shape=jax.ShapeDtypeStruct((_NVPAD,), jnp.float32),
    )(feats, W, b2d)


_CH = 3200
_CPR = 8
_LAST_COL = P - _CH


@functools.lru_cache(maxsize=1)
def _make_sc_gather():
    mesh = plsc.VectorSubcoreMesh(core_axis_name="c", subcore_axis_name="s")

    @functools.partial(
        pl.kernel,
        mesh=mesh,
        out_type=jax.ShapeDtypeStruct((B * P,), jnp.float32),
        scratch_types=[
            pltpu.VMEM((_CH,), jnp.int32),
            pltpu.VMEM((_CH,), jnp.float32),
            pltpu.SemaphoreType.DMA,
        ],
    )
    def gather_k(scores_hbm, idx_hbm, out_hbm, idx_v, vals_v, sem):
        wid = lax.axis_index("s") * 2 + lax.axis_index("c")
        base = jnp.minimum(wid * _CH, B * P - _CH)
        pltpu.sync_copy(idx_hbm.at[pl.ds(base, _CH)], idx_v)
        pltpu.async_copy(scores_hbm.at[idx_v], vals_v, sem).wait()
        pltpu.sync_copy(vals_v, out_hbm.at[pl.ds(base, _CH)])

    return gather_k


def kernel(feats_backbone, quantize2original, W, b):
    scores = _voxel_scores(feats_backbone, W, b.reshape(1, 1))
    return _make_sc_gather()(scores, quantize2original).reshape(B, P)
